# baseline (speedup 1.0000x reference)
import functools
import math

import jax
import jax.numpy as jnp
from jax import lax
from jax.experimental import pallas as pl
from jax.experimental.pallas import tpu as pltpu
from jax.experimental.pallas import tpu_sc as plsc

N = 10000
E = 160000
K = 5000
D = 128

NP = 10112
NW = 32
NCHUNK = 40
CL = 128
EP = NW * NCHUNK * CL
ROWS_PER_TILE = NP // 16
PAD_ROW = 10048


def _make_edge_pass(C):
    mesh = plsc.VectorSubcoreMesh(core_axis_name="c", subcore_axis_name="s")

    @functools.partial(
        pl.kernel,
        mesh=mesh,
        out_type=jax.ShapeDtypeStruct((2, NP, C), jnp.float32),
        scratch_types=[
            pltpu.VMEM((CL,), jnp.int32),
            pltpu.VMEM((CL,), jnp.int32),
            pltpu.VMEM((CL, C), jnp.float32),
            pltpu.VMEM_SHARED((NP, C), jnp.float32),
            pltpu.SemaphoreType.DMA,
        ],
    )
    def kern(table, gidx, sidx, zeros, out, gv, sv, rows, accum, sem):
        cid = lax.axis_index("c")
        sid = lax.axis_index("s")
        wid = sid * 2 + cid
        r0 = sid * ROWS_PER_TILE
        pltpu.sync_copy(zeros.at[pl.ds(r0, ROWS_PER_TILE)],
                        accum.at[pl.ds(r0, ROWS_PER_TILE)])
        plsc.subcore_barrier()

        def body(j, carry):
            pltpu.sync_copy(gidx.at[wid, j], gv)
            pltpu.sync_copy(sidx.at[wid, j], sv)
            pltpu.async_copy(table.at[gv], rows, sem).wait()
            pltpu.sync_copy(rows, accum.at[sv], add=True)
            return carry

        lax.fori_loop(0, NCHUNK, body, 0)
        plsc.subcore_barrier()
        pltpu.sync_copy(accum.at[pl.ds(r0, ROWS_PER_TILE)],
                        out.at[cid, pl.ds(r0, ROWS_PER_TILE)])

    return kern


_edge_pass_feat = _make_edge_pass(D)


def _mm_kernel(x_ref, w_ref, is_ref, os_ref, o_ref):
    o_ref[...] = os_ref[...] * jnp.dot(
        is_ref[...] * x_ref[...], w_ref[...],
        preferred_element_type=jnp.float32)


def _mm_scale(x, w, iscale, oscale):
    B = 512
    return pl.pallas_call(
        _mm_kernel,
        grid=(pl.cdiv(NP, B),),
        in_specs=[
            pl.BlockSpec((B, D), lambda i: (i, 0)),
            pl.BlockSpec((D, D), lambda i: (0, 0)),
            pl.BlockSpec((B, D), lambda i: (i, 0)),
            pl.BlockSpec((B, D), lambda i: (i, 0)),
        ],
        out_specs=pl.BlockSpec((B, D), lambda i: (i, 0)),
        out_shape=jax.ShapeDtypeStruct((NP, D), jnp.float32),
    )(x, w, iscale, oscale)


def _make_ewise(relu):
    def _ew_kernel(a_ref, b_ref, z_ref, d_ref, bias_ref, o_ref):
        r = d_ref[...] * (a_ref[...] + b_ref[...] + 2.0 * z_ref[...]) + bias_ref[...]
        if relu:
            r = jnp.maximum(r, 0.0)
        o_ref[...] = r

    def run(a, b, z, dscale, bias):
        B = 512
        return pl.pallas_call(
            _ew_kernel,
            grid=(pl.cdiv(NP, B),),
            in_specs=[pl.BlockSpec((B, D), lambda i: (i, 0))] * 5,
            out_specs=pl.BlockSpec((B, D), lambda i: (i, 0)),
            out_shape=jax.ShapeDtypeStruct((NP, D), jnp.float32),
        )(a, b, z, dscale, bias)

    return run


_conv_epilogue_relu = _make_ewise(True)
_conv_epilogue = _make_ewise(False)


def _bc(col):
    return jnp.broadcast_to(col[:, None], (NP, D))


def kernel(x, edge_index, conv0_w, conv0_b, conv1_w, conv1_b, pool0_p, up0_w, up0_b):
    src = edge_index[0]
    dst = edge_index[1]

    xp = jnp.zeros((NP, D), jnp.float32).at[:N].set(x)
    pad_e = jnp.full((EP - E,), PAD_ROW, jnp.int32)
    srcp = jnp.concatenate([src, pad_e]).reshape(NW, NCHUNK, CL)
    dstp = jnp.concatenate([dst, pad_e]).reshape(NW, NCHUNK, CL)
    zeros128 = jnp.zeros((NP, D), jnp.float32)

    keys = dst * N + src
    q = src * N + dst
    sk = jnp.sort(keys)
    r = (jnp.searchsorted(sk, q, side='right')
         - jnp.searchsorted(sk, q, side='left')).astype(jnp.float32)
    selfloop = (src == dst).astype(jnp.float32)

    payload = jnp.zeros((EP, D), jnp.float32)
    payload = payload.at[:E, 0].set(1.0)
    payload = payload.at[:E, 1].set(selfloop)
    payload = payload.at[:E, 2].set(r * (1.0 - selfloop))
    iota_e = jnp.arange(EP, dtype=jnp.int32).reshape(NW, NCHUNK, CL)
    p0 = _edge_pass_feat(payload, iota_e, dstp, zeros128)
    acc0 = p0[0] + p0[1]
    degA, sl, c = acc0[:, 0], acc0[:, 1], acc0[:, 2]

    dinv = 1.0 / jnp.sqrt(degA + 2.0)
    dinvB = _bc(dinv)
    onesB = jnp.ones((NP, D), jnp.float32)
    slB = _bc(sl)

    z1 = _mm_scale(xp, conv0_w, onesB, dinvB)
    p1 = _edge_pass_feat(z1, srcp, dstp, zeros128)
    b0B = jnp.broadcast_to(conv0_b[None, :], (NP, D))
    x1 = _conv_epilogue_relu(p1[0], p1[1], z1, dinvB, b0B)
    x1 = x1.at[N:].set(0.0)

    score = jnp.tanh((x1[:N] @ pool0_p) / jnp.linalg.norm(pool0_p))
    kth = lax.top_k(score, K)[0][-1]
    gt = score > kth
    ngt = jnp.sum(gt.astype(jnp.int32))
    ties = score == kth
    tiecum = jnp.cumsum(ties.astype(jnp.int32))
    s_mask = (gt | (ties & (tiecum <= (K - ngt)))).astype(jnp.float32)
    s_mask = jnp.zeros((NP,), jnp.float32).at[:N].set(s_mask)
    scoref = jnp.zeros((NP,), jnp.float32).at[:N].set(score)

    t_in = jnp.zeros((NP, D), jnp.float32).at[:, 0].set(s_mask)
    p2 = _edge_pass_feat(t_in, srcp, dstp, zeros128)
    t0 = (p2[0] + p2[1])[:, 0] + s_mask - sl * s_mask
    t_in2 = jnp.zeros((NP, D), jnp.float32).at[:, 0].set(t0)
    p3 = _edge_pass_feat(t_in2, srcp, dstp, zeros128)
    u0 = (p3[0] + p3[1])[:, 0] + t0 - sl * t0
    deg2 = u0 + 1.0 - c
    d2 = s_mask / jnp.sqrt(jnp.maximum(deg2, 1.0))
    d2B = _bc(d2)

    msB = _bc(s_mask * scoref)
    v = _mm_scale(x1, conv1_w, msB, d2B)
    p4 = _edge_pass_feat(v, srcp, dstp, zeros128)
    tv = p4[0] + p4[1] + v - slB * v
    p5 = _edge_pass_feat(tv, srcp, dstp, zeros128)
    uv = p5[0] + p5[1] + tv - slB * tv
    sB = _bc(s_mask)
    cB = _bc(c)
    out2 = sB * jax.nn.relu(d2B * (sB * uv + (1.0 - cB) * v) + conv1_b[None, :])

    x3 = x1 + out2
    z3 = _mm_scale(x3, up0_w, onesB, dinvB)
    p6 = _edge_pass_feat(z3, srcp, dstp, zeros128)
    buB = jnp.broadcast_to(up0_b[None, :], (NP, D))
    out = _conv_epilogue(p6[0], p6[1], z3, dinvB, buB)
    return out[:N]


# speedup vs baseline: 1.0412x; 1.0412x over previous
import functools
import math

import jax
import jax.numpy as jnp
from jax import lax
from jax.experimental import pallas as pl
from jax.experimental.pallas import tpu as pltpu
from jax.experimental.pallas import tpu_sc as plsc

N = 10000
E = 160000
K = 5000
D = 128

NP = 10112
NW = 32
NCHUNK = 40
CL = 128
EP = NW * NCHUNK * CL
ROWS_PER_TILE = NP // 16
PAD_ROW = 10048


def _make_edge_pass(C):
    mesh = plsc.VectorSubcoreMesh(core_axis_name="c", subcore_axis_name="s")

    @functools.partial(
        pl.kernel,
        mesh=mesh,
        out_type=jax.ShapeDtypeStruct((2, NP, C), jnp.float32),
        scratch_types=[
            pltpu.VMEM((NCHUNK, CL), jnp.int32),
            pltpu.VMEM((NCHUNK, CL), jnp.int32),
            pltpu.VMEM((CL, C), jnp.float32),
            pltpu.VMEM((CL, C), jnp.float32),
            pltpu.VMEM_SHARED((NP, C), jnp.float32),
            pltpu.SemaphoreType.DMA,
            pltpu.SemaphoreType.DMA,
        ],
    )
    def kern(table, gidx, sidx, zeros, out, gv, sv, rows0, rows1, accum,
             sem0, sem1):
        cid = lax.axis_index("c")
        sid = lax.axis_index("s")
        wid = sid * 2 + cid
        r0 = sid * ROWS_PER_TILE
        pltpu.sync_copy(gidx.at[wid], gv)
        pltpu.sync_copy(sidx.at[wid], sv)
        pltpu.sync_copy(zeros.at[pl.ds(r0, ROWS_PER_TILE)],
                        accum.at[pl.ds(r0, ROWS_PER_TILE)])
        plsc.subcore_barrier()

        def body(i, carry):
            j0 = 2 * i
            c0 = pltpu.async_copy(table.at[gv.at[j0]], rows0, sem0)
            c1 = pltpu.async_copy(table.at[gv.at[j0 + 1]], rows1, sem1)
            c0.wait()
            pltpu.sync_copy(rows0, accum.at[sv.at[j0]], add=True)
            c1.wait()
            pltpu.sync_copy(rows1, accum.at[sv.at[j0 + 1]], add=True)
            return carry

        lax.fori_loop(0, NCHUNK // 2, body, 0)
        plsc.subcore_barrier()
        pltpu.sync_copy(accum.at[pl.ds(r0, ROWS_PER_TILE)],
                        out.at[cid, pl.ds(r0, ROWS_PER_TILE)])

    return kern


_edge_pass_feat = _make_edge_pass(D)


def _mm_kernel(x_ref, w_ref, is_ref, os_ref, o_ref):
    o_ref[...] = os_ref[...] * jnp.dot(
        is_ref[...] * x_ref[...], w_ref[...],
        preferred_element_type=jnp.float32)


def _mm_scale(x, w, iscale, oscale):
    B = 512
    return pl.pallas_call(
        _mm_kernel,
        grid=(pl.cdiv(NP, B),),
        in_specs=[
            pl.BlockSpec((B, D), lambda i: (i, 0)),
            pl.BlockSpec((D, D), lambda i: (0, 0)),
            pl.BlockSpec((B, D), lambda i: (i, 0)),
            pl.BlockSpec((B, D), lambda i: (i, 0)),
        ],
        out_specs=pl.BlockSpec((B, D), lambda i: (i, 0)),
        out_shape=jax.ShapeDtypeStruct((NP, D), jnp.float32),
    )(x, w, iscale, oscale)


def _make_ewise(relu):
    def _ew_kernel(a_ref, b_ref, z_ref, d_ref, bias_ref, o_ref):
        r = d_ref[...] * (a_ref[...] + b_ref[...] + 2.0 * z_ref[...]) + bias_ref[...]
        if relu:
            r = jnp.maximum(r, 0.0)
        o_ref[...] = r

    def run(a, b, z, dscale, bias):
        B = 512
        return pl.pallas_call(
            _ew_kernel,
            grid=(pl.cdiv(NP, B),),
            in_specs=[pl.BlockSpec((B, D), lambda i: (i, 0))] * 5,
            out_specs=pl.BlockSpec((B, D), lambda i: (i, 0)),
            out_shape=jax.ShapeDtypeStruct((NP, D), jnp.float32),
        )(a, b, z, dscale, bias)

    return run


_conv_epilogue_relu = _make_ewise(True)
_conv_epilogue = _make_ewise(False)


def _bc(col):
    return jnp.broadcast_to(col[:, None], (NP, D))


def kernel(x, edge_index, conv0_w, conv0_b, conv1_w, conv1_b, pool0_p, up0_w, up0_b):
    src = edge_index[0]
    dst = edge_index[1]

    xp = jnp.zeros((NP, D), jnp.float32).at[:N].set(x)
    pad_e = jnp.full((EP - E,), PAD_ROW, jnp.int32)
    srcp = jnp.concatenate([src, pad_e]).reshape(NW, NCHUNK, CL)
    dstp = jnp.concatenate([dst, pad_e]).reshape(NW, NCHUNK, CL)
    zeros128 = jnp.zeros((NP, D), jnp.float32)

    keys = dst * N + src
    q = src * N + dst
    sk = jnp.sort(keys)
    r = (jnp.searchsorted(sk, q, side='right')
         - jnp.searchsorted(sk, q, side='left')).astype(jnp.float32)
    selfloop = (src == dst).astype(jnp.float32)

    payload = jnp.zeros((EP, D), jnp.float32)
    payload = payload.at[:E, 0].set(1.0)
    payload = payload.at[:E, 1].set(selfloop)
    payload = payload.at[:E, 2].set(r * (1.0 - selfloop))
    iota_e = jnp.arange(EP, dtype=jnp.int32).reshape(NW, NCHUNK, CL)
    p0 = _edge_pass_feat(payload, iota_e, dstp, zeros128)
    acc0 = p0[0] + p0[1]
    degA, sl, c = acc0[:, 0], acc0[:, 1], acc0[:, 2]

    dinv = 1.0 / jnp.sqrt(degA + 2.0)
    dinvB = _bc(dinv)
    onesB = jnp.ones((NP, D), jnp.float32)
    slB = _bc(sl)

    z1 = _mm_scale(xp, conv0_w, onesB, dinvB)
    p1 = _edge_pass_feat(z1, srcp, dstp, zeros128)
    b0B = jnp.broadcast_to(conv0_b[None, :], (NP, D))
    x1 = _conv_epilogue_relu(p1[0], p1[1], z1, dinvB, b0B)
    x1 = x1.at[N:].set(0.0)

    score = jnp.tanh((x1[:N] @ pool0_p) / jnp.linalg.norm(pool0_p))
    kth = lax.top_k(score, K)[0][-1]
    gt = score > kth
    ngt = jnp.sum(gt.astype(jnp.int32))
    ties = score == kth
    tiecum = jnp.cumsum(ties.astype(jnp.int32))
    s_mask = (gt | (ties & (tiecum <= (K - ngt)))).astype(jnp.float32)
    s_mask = jnp.zeros((NP,), jnp.float32).at[:N].set(s_mask)
    scoref = jnp.zeros((NP,), jnp.float32).at[:N].set(score)

    t_in = jnp.zeros((NP, D), jnp.float32).at[:, 0].set(s_mask)
    p2 = _edge_pass_feat(t_in, srcp, dstp, zeros128)
    t0 = (p2[0] + p2[1])[:, 0] + s_mask - sl * s_mask
    t_in2 = jnp.zeros((NP, D), jnp.float32).at[:, 0].set(t0)
    p3 = _edge_pass_feat(t_in2, srcp, dstp, zeros128)
    u0 = (p3[0] + p3[1])[:, 0] + t0 - sl * t0
    deg2 = u0 + 1.0 - c
    d2 = s_mask / jnp.sqrt(jnp.maximum(deg2, 1.0))
    d2B = _bc(d2)

    msB = _bc(s_mask * scoref)
    v = _mm_scale(x1, conv1_w, msB, d2B)
    p4 = _edge_pass_feat(v, srcp, dstp, zeros128)
    tv = p4[0] + p4[1] + v - slB * v
    p5 = _edge_pass_feat(tv, srcp, dstp, zeros128)
    uv = p5[0] + p5[1] + tv - slB * tv
    sB = _bc(s_mask)
    cB = _bc(c)
    out2 = sB * jax.nn.relu(d2B * (sB * uv + (1.0 - cB) * v) + conv1_b[None, :])

    x3 = x1 + out2
    z3 = _mm_scale(x3, up0_w, onesB, dinvB)
    p6 = _edge_pass_feat(z3, srcp, dstp, zeros128)
    buB = jnp.broadcast_to(up0_b[None, :], (NP, D))
    out = _conv_epilogue(p6[0], p6[1], z3, dinvB, buB)
    return out[:N]
